# Initial kernel scaffold; baseline (speedup 1.0000x reference)
import functools

import jax
import jax.numpy as jnp
from jax import lax
from jax.experimental import pallas as pl
from jax.experimental.pallas import tpu as pltpu
from jax.experimental.pallas import tpu_sc as plsc

N = 10000
E = 320000
D = 128
DE = 16
G = 512
T = 10

NC = 2
NS = 16
NW = NC * NS
EPW = E // NW
CH = 80
NCHUNK = EPW // CH
RPT = N // NS
RCH = 125
LANES = 16


_EB = 8000


def _encode_body(attr_ref, we_ref, out_ref):
    out_ref[...] = jnp.dot(attr_ref[...], we_ref[...],
                           preferred_element_type=jnp.float32)


def _encode(edge_attr, W_e):
    return pl.pallas_call(
        _encode_body,
        grid=(E // _EB,),
        in_specs=[
            pl.BlockSpec((_EB, DE), lambda i: (i, 0)),
            pl.BlockSpec((DE, D), lambda i: (0, 0)),
        ],
        out_specs=pl.BlockSpec((_EB, D), lambda i: (i, 0)),
        out_shape=jax.ShapeDtypeStruct((E, D), jnp.float32),
    )(edge_attr, W_e)


def _edge_sc_body(x_hbm, src_hbm, dst_hbm, m_hbm, out_hbm,
                  src_v, dst_v, xrows_v, mrows_v, zbuf_v, acc_sh, gsem):
    c = lax.axis_index("c")
    s = lax.axis_index("s")
    wid = s * NC + c

    def zrow(i, _):
        for j in range(D // LANES):
            zbuf_v[i, pl.ds(j * LANES, LANES)] = jnp.zeros((LANES,), jnp.float32)
        return 0
    lax.fori_loop(0, RCH, zrow, 0)

    def zcp(i, _):
        pltpu.sync_copy(zbuf_v, acc_sh.at[pl.ds(s * RPT + i * RCH, RCH), :])
        return 0
    lax.fori_loop(0, RPT // RCH, zcp, 0)
    plsc.subcore_barrier()

    def chunk(ci, _):
        base = wid * EPW + ci * CH
        pltpu.sync_copy(src_hbm.at[pl.ds(base, CH)], src_v)
        pltpu.sync_copy(dst_hbm.at[pl.ds(base, CH)], dst_v)
        gather = pltpu.async_copy(x_hbm.at[src_v], xrows_v, gsem)
        pltpu.sync_copy(m_hbm.at[pl.ds(base, CH), :], mrows_v)
        gather.wait()

        def row(e, _):
            for j in range(D // LANES):
                sl = pl.ds(j * LANES, LANES)
                mrows_v[e, sl] = jnp.maximum(xrows_v[e, sl] + mrows_v[e, sl],
                                             0.0)
            return 0
        lax.fori_loop(0, CH, row, 0)

        pltpu.sync_copy(mrows_v, acc_sh.at[dst_v], add=True)
        return 0
    lax.fori_loop(0, NCHUNK, chunk, 0)
    plsc.subcore_barrier()

    def wout(i, _):
        r0 = s * RPT + i * RCH
        pltpu.sync_copy(acc_sh.at[pl.ds(r0, RCH), :], zbuf_v)
        pltpu.sync_copy(zbuf_v, out_hbm.at[c, pl.ds(r0, RCH), :])
        return 0
    lax.fori_loop(0, RPT // RCH, wout, 0)


def _edge_sc(x, src, dst, M):
    mesh = plsc.VectorSubcoreMesh(core_axis_name="c", subcore_axis_name="s",
                                  num_cores=NC, num_subcores=NS)
    return pl.kernel(
        _edge_sc_body,
        out_type=jax.ShapeDtypeStruct((NC, N, D), jnp.float32),
        mesh=mesh,
        scratch_types=[
            pltpu.VMEM((CH,), jnp.int32),
            pltpu.VMEM((CH,), jnp.int32),
            pltpu.VMEM((CH, D), jnp.float32),
            pltpu.VMEM((CH, D), jnp.float32),
            pltpu.VMEM((RCH, D), jnp.float32),
            pltpu.VMEM_SHARED((N, D), jnp.float32),
            pltpu.SemaphoreType.DMA,
        ],
    )(x, src, dst, M)


_R = 2000


def _finish_body(agg_ref, x_ref, b_ref, w1_ref, b1_ref, wh_ref, bh_ref,
                 out_ref, sums_ref, counts_ref):
    i = pl.program_id(0)

    @pl.when(i == 0)
    def _():
        sums_ref[...] = jnp.zeros_like(sums_ref)
        counts_ref[...] = jnp.zeros_like(counts_ref)

    z = agg_ref[0] + agg_ref[1] + x_ref[...]
    h = jnp.maximum(
        jnp.dot(z, w1_ref[...], preferred_element_type=jnp.float32)
        + b1_ref[...], 0.0)
    bids = b_ref[0, 0, :]
    gi = lax.broadcasted_iota(jnp.int32, (G, _R), 0)
    oh = (gi == bids[None, :]).astype(jnp.float32)
    sums_ref[...] += jnp.dot(oh, h, preferred_element_type=jnp.float32)
    counts_ref[...] += jnp.sum(oh, axis=1)[None, :]

    @pl.when(i == pl.num_programs(0) - 1)
    def _():
        pooled = sums_ref[...] / jnp.maximum(counts_ref[0, :], 1.0)[:, None]
        out_ref[...] = (jnp.dot(pooled, wh_ref[...],
                                preferred_element_type=jnp.float32)
                        + bh_ref[...])


def _finish(agg2, x, batch3d, W1, b1, W_head, b_head):
    nblk = N // _R
    return pl.pallas_call(
        _finish_body,
        grid=(nblk,),
        in_specs=[
            pl.BlockSpec((NC, _R, D), lambda i: (0, i, 0)),
            pl.BlockSpec((_R, D), lambda i: (i, 0)),
            pl.BlockSpec((1, 1, _R), lambda i: (i, 0, 0)),
            pl.BlockSpec((D, D), lambda i: (0, 0)),
            pl.BlockSpec((1, D), lambda i: (0, 0)),
            pl.BlockSpec((D, T), lambda i: (0, 0)),
            pl.BlockSpec((1, T), lambda i: (0, 0)),
        ],
        out_specs=pl.BlockSpec((G, T), lambda i: (0, 0)),
        out_shape=jax.ShapeDtypeStruct((G, T), jnp.float32),
        scratch_shapes=[
            pltpu.VMEM((G, D), jnp.float32),
            pltpu.VMEM((1, G), jnp.float32),
        ],
    )(agg2, x, batch3d, W1, b1, W_head, b_head)


def kernel(x, edge_index, edge_attr, batch_assignments, W_e, W1, b1,
           W_head, b_head):
    src = edge_index[0]
    dst = edge_index[1]
    M = _encode(edge_attr, W_e)
    agg2 = _edge_sc(x, src, dst, M)
    batch3d = batch_assignments.reshape(N // _R, 1, _R)
    out = _finish(agg2, x, batch3d, W1, b1.reshape(1, D),
                  W_head, b_head.reshape(1, T))
    return out


# speedup vs baseline: 3.1190x; 3.1190x over previous
import functools

import jax
import jax.numpy as jnp
from jax import lax
from jax.experimental import pallas as pl
from jax.experimental.pallas import tpu as pltpu
from jax.experimental.pallas import tpu_sc as plsc

N = 10000
E = 320000
D = 128
DE = 16
G = 512
T = 10

NC = 2
NS = 16
NW = NC * NS
EPW = E // NW
CH = 80
NCHUNK = EPW // CH
NP = 10240
RPT = NP // NS
RCH = 128
LANES = 16


_EB = 8000


def _encode_body(attr_ref, we_ref, out_ref):
    out_ref[...] = jnp.dot(attr_ref[...], we_ref[...],
                           preferred_element_type=jnp.float32)


def _encode(edge_attr, W_e):
    return pl.pallas_call(
        _encode_body,
        grid=(E // _EB,),
        in_specs=[
            pl.BlockSpec((_EB, DE), lambda i: (i, 0)),
            pl.BlockSpec((DE, D), lambda i: (0, 0)),
        ],
        out_specs=pl.BlockSpec((_EB, D), lambda i: (i, 0)),
        out_shape=jax.ShapeDtypeStruct((E, D), jnp.float32),
    )(edge_attr, W_e)


def _edge_sc_body(x_hbm, src_hbm, dst_hbm, m_hbm, out_hbm,
                  src_v, dst_v, xrows_v, mrows_v, zbuf_v, acc_sh, gsem):
    c = lax.axis_index("c")
    s = lax.axis_index("s")
    wid = s * NC + c

    def zrow(i, _):
        for j in range(D // LANES):
            zbuf_v[i, pl.ds(j * LANES, LANES)] = jnp.zeros((LANES,), jnp.float32)
        return 0
    lax.fori_loop(0, RCH, zrow, 0)

    def zcp(i, _):
        pltpu.sync_copy(zbuf_v, acc_sh.at[pl.ds(s * RPT + i * RCH, RCH), :])
        return 0
    lax.fori_loop(0, RPT // RCH, zcp, 0)
    plsc.subcore_barrier()

    def chunk(ci, _):
        base = wid * EPW + ci * CH
        pltpu.sync_copy(src_hbm.at[pl.ds(base, CH)], src_v)
        pltpu.sync_copy(dst_hbm.at[pl.ds(base, CH)], dst_v)
        gather = pltpu.async_copy(x_hbm.at[src_v], xrows_v, gsem)
        pltpu.sync_copy(m_hbm.at[pl.ds(base, CH), :], mrows_v)
        gather.wait()

        def row(e, _):
            for j in range(D // LANES):
                sl = pl.ds(j * LANES, LANES)
                mrows_v[e, sl] = jnp.maximum(xrows_v[e, sl] + mrows_v[e, sl],
                                             0.0)
            return 0
        lax.fori_loop(0, CH, row, 0)

        pltpu.sync_copy(mrows_v, acc_sh.at[dst_v], add=True)
        return 0
    lax.fori_loop(0, NCHUNK, chunk, 0)
    plsc.subcore_barrier()

    def wout(i, _):
        r0 = s * RPT + i * RCH
        pltpu.sync_copy(acc_sh.at[pl.ds(r0, RCH), :], zbuf_v)
        pltpu.sync_copy(zbuf_v, out_hbm.at[c, pl.ds(r0, RCH), :])
        return 0
    lax.fori_loop(0, RPT // RCH, wout, 0)


def _edge_sc(x, src, dst, M):
    mesh = plsc.VectorSubcoreMesh(core_axis_name="c", subcore_axis_name="s",
                                  num_cores=NC, num_subcores=NS)
    return pl.kernel(
        _edge_sc_body,
        out_type=jax.ShapeDtypeStruct((NC, NP, D), jnp.float32),
        mesh=mesh,
        scratch_types=[
            pltpu.VMEM((CH,), jnp.int32),
            pltpu.VMEM((CH,), jnp.int32),
            pltpu.VMEM((CH, D), jnp.float32),
            pltpu.VMEM((CH, D), jnp.float32),
            pltpu.VMEM((RCH, D), jnp.float32),
            pltpu.VMEM_SHARED((NP, D), jnp.float32),
            pltpu.SemaphoreType.DMA,
        ],
    )(x, src, dst, M)


_R = 2000


def _finish_body(agg_ref, x_ref, b_ref, w1_ref, b1_ref, wh_ref, bh_ref,
                 out_ref, sums_ref, counts_ref):
    i = pl.program_id(0)

    @pl.when(i == 0)
    def _():
        sums_ref[...] = jnp.zeros_like(sums_ref)
        counts_ref[...] = jnp.zeros_like(counts_ref)

    z = agg_ref[0] + agg_ref[1] + x_ref[...]
    h = jnp.maximum(
        jnp.dot(z, w1_ref[...], preferred_element_type=jnp.float32)
        + b1_ref[...], 0.0)
    bids = b_ref[0, 0, :]
    gi = lax.broadcasted_iota(jnp.int32, (G, _R), 0)
    oh = (gi == bids[None, :]).astype(jnp.float32)
    sums_ref[...] += jnp.dot(oh, h, preferred_element_type=jnp.float32)
    counts_ref[...] += jnp.sum(oh, axis=1)[None, :]

    @pl.when(i == pl.num_programs(0) - 1)
    def _():
        pooled = sums_ref[...] / jnp.maximum(counts_ref[0, :], 1.0)[:, None]
        out_ref[...] = (jnp.dot(pooled, wh_ref[...],
                                preferred_element_type=jnp.float32)
                        + bh_ref[...])


def _finish(agg2, x, batch3d, W1, b1, W_head, b_head):
    nblk = N // _R
    return pl.pallas_call(
        _finish_body,
        grid=(nblk,),
        in_specs=[
            pl.BlockSpec((NC, _R, D), lambda i: (0, i, 0)),
            pl.BlockSpec((_R, D), lambda i: (i, 0)),
            pl.BlockSpec((1, 1, _R), lambda i: (i, 0, 0)),
            pl.BlockSpec((D, D), lambda i: (0, 0)),
            pl.BlockSpec((1, D), lambda i: (0, 0)),
            pl.BlockSpec((D, T), lambda i: (0, 0)),
            pl.BlockSpec((1, T), lambda i: (0, 0)),
        ],
        out_specs=pl.BlockSpec((G, T), lambda i: (0, 0)),
        out_shape=jax.ShapeDtypeStruct((G, T), jnp.float32),
        scratch_shapes=[
            pltpu.VMEM((G, D), jnp.float32),
            pltpu.VMEM((1, G), jnp.float32),
        ],
    )(agg2, x, batch3d, W1, b1, W_head, b_head)


def kernel(x, edge_index, edge_attr, batch_assignments, W_e, W1, b1,
           W_head, b_head):
    src = edge_index[0]
    dst = edge_index[1]
    M = _encode(edge_attr, W_e)
    agg2 = _edge_sc(x, src, dst, M)
    batch3d = batch_assignments.reshape(N // _R, 1, _R)
    out = _finish(agg2, x, batch3d, W1, b1.reshape(1, D),
                  W_head, b_head.reshape(1, T))
    return out
